# Initial kernel scaffold; baseline (speedup 1.0000x reference)
import jax
import jax.numpy as jnp
from jax.experimental import pallas as pl


def kernel(feats, batch, W, b):
    raise NotImplementedError("write your pallas kernel here")


# speedup vs baseline: 2.0868x; 2.0868x over previous
import functools

import jax
import jax.numpy as jnp
from jax import lax
from jax.experimental import pallas as pl
from jax.experimental.pallas import tpu as pltpu
from jax.experimental.pallas import tpu_sc as plsc

N, D, S = 320000, 128, 10000
NC, NS, L = 2, 16, 16
NW = NC * NS
RW = N // NW
BLK = 200
NBLK = RW // BLK
SCW = 100
NSCAT = BLK // SCW
IROWS = RW // SCW
SROWS = 624
TAIL_OFF = SROWS * NS
TAIL = S - TAIL_OFF

WBLK = 512


def _weights_body(f_ref, w_ref, b_ref, o_ref):
    f = f_ref[...]
    logits = jnp.sum(f * w_ref[...], axis=1) + b_ref[0, 0]
    o_ref[...] = f * jax.nn.sigmoid(logits)[:, None]


def _row_weights(feats, W, b):
    return pl.pallas_call(
        _weights_body,
        grid=(N // WBLK,),
        in_specs=[
            pl.BlockSpec((WBLK, D), lambda i: (i, 0)),
            pl.BlockSpec((1, D), lambda i: (0, 0)),
            pl.BlockSpec(memory_space=pltpu.SMEM),
        ],
        out_specs=pl.BlockSpec((WBLK, D), lambda i: (i, 0)),
        out_shape=jax.ShapeDtypeStruct((N, D), jnp.float32),
    )(feats, W, b)


def _sc_pool(batch2d, w, zeros):
    mesh = plsc.VectorSubcoreMesh(
        core_axis_name="c", subcore_axis_name="s",
        num_cores=NC, num_subcores=NS)

    @functools.partial(
        pl.kernel,
        out_type=jax.ShapeDtypeStruct((NC, S, D), jnp.float32),
        mesh=mesh,
        compiler_params=pltpu.CompilerParams(use_tc_tiling_on_sc=False),
        scratch_types=[
            pltpu.VMEM((BLK, D), jnp.float32),
            pltpu.VMEM((IROWS, SCW), jnp.int32),
            pltpu.VMEM_SHARED((S, D), jnp.float32),
        ],
    )
    def k(batch_hbm, w_hbm, z_hbm, out_hbm, fbuf, ibuf, acc):
        c = lax.axis_index("c")
        s = lax.axis_index("s")
        wid = c * NS + s
        base = wid * RW

        pltpu.sync_copy(batch_hbm.at[pl.ds(wid * IROWS, IROWS), :], ibuf)

        pltpu.sync_copy(z_hbm.at[pl.ds(s * SROWS, SROWS), :],
                        acc.at[pl.ds(s * SROWS, SROWS), :])

        @pl.when(s == 0)
        def _zero_tail():
            pltpu.sync_copy(z_hbm.at[pl.ds(TAIL_OFF, TAIL), :],
                            acc.at[pl.ds(TAIL_OFF, TAIL), :])
        plsc.subcore_barrier()

        def blk_body(i, carry):
            r0 = pl.multiple_of(base + i * BLK, 8)
            pltpu.sync_copy(w_hbm.at[pl.ds(r0, BLK), :], fbuf)

            for cc in range(NSCAT):
                pltpu.sync_copy(fbuf.at[pl.ds(cc * SCW, SCW), :],
                                acc.at[ibuf.at[i * NSCAT + cc]], add=True)
            return carry
        lax.fori_loop(0, NBLK, blk_body, 0)

        plsc.subcore_barrier()
        pltpu.sync_copy(acc.at[pl.ds(s * SROWS, SROWS), :],
                        out_hbm.at[c, pl.ds(s * SROWS, SROWS), :])

        @pl.when(s == 0)
        def _drain_tail():
            pltpu.sync_copy(acc.at[pl.ds(TAIL_OFF, TAIL), :],
                            out_hbm.at[c, pl.ds(TAIL_OFF, TAIL), :])

    return k(batch2d, w, zeros)


def _combine_body(p_ref, o_ref):
    o_ref[...] = p_ref[0] + p_ref[1]


def _combine(parts):
    CB = 1000
    return pl.pallas_call(
        _combine_body,
        grid=(S // CB,),
        in_specs=[pl.BlockSpec((NC, CB, D), lambda i: (0, i, 0))],
        out_specs=pl.BlockSpec((CB, D), lambda i: (i, 0)),
        out_shape=jax.ShapeDtypeStruct((S, D), jnp.float32),
    )(parts)


def kernel(feats, batch, W, b):
    w = _row_weights(feats, W.reshape(1, D), b.reshape(1, 1))
    parts = _sc_pool(batch.reshape(N // SCW, SCW), w,
                     jnp.zeros((S, D), jnp.float32))
    return _combine(parts)
